# Initial kernel scaffold; baseline (speedup 1.0000x reference)
import jax
import jax.numpy as jnp
from jax.experimental import pallas as pl


def kernel(x, edge_index, W1, b1, W2, b2):
    raise NotImplementedError("write your pallas kernel here")


# speedup vs baseline: 14.7225x; 14.7225x over previous
import functools

import jax
import jax.numpy as jnp
from jax import lax
from jax.experimental import pallas as pl
from jax.experimental.pallas import tpu as pltpu
from jax.experimental.pallas import tpu_sc as plsc

NC = 2
NS = 16
NW = NC * NS
CHUNK = 80


def _mesh():
    return plsc.VectorSubcoreMesh(core_axis_name="c", subcore_axis_name="s")


def _pad_rows(N):
    r = -(-N // NS)
    return -(-r // 8) * 8


def _deg_call(E, N, DC):
    e_per_w = E // NW
    n_chunks = e_per_w // CHUNK
    rows_per_tile = _pad_rows(N)
    NP = rows_per_tile * NS

    @functools.partial(
        pl.kernel,
        mesh=_mesh(),
        out_type=jax.ShapeDtypeStruct((NC, NP, DC), jnp.float32),
        compiler_params=pltpu.CompilerParams(use_tc_tiling_on_sc=False),
        scratch_types=[
            pltpu.VMEM((CHUNK,), jnp.int32),
            pltpu.VMEM((CHUNK, DC), jnp.float32),
            pltpu.VMEM_SHARED((NP, DC), jnp.float32),
        ],
    )
    def k(dst_hbm, ones_hbm, zeros_hbm, out_hbm, idx_v, ones_v, acc):
        c = lax.axis_index("c")
        s = lax.axis_index("s")
        wid = c * NS + s
        rbase = s * rows_per_tile
        pltpu.sync_copy(zeros_hbm, acc.at[pl.ds(rbase, rows_per_tile)])
        pltpu.sync_copy(ones_hbm, ones_v)
        plsc.subcore_barrier()
        ebase = wid * e_per_w

        def body(i, carry):
            off = ebase + i * CHUNK
            pltpu.sync_copy(dst_hbm.at[pl.ds(off, CHUNK)], idx_v)
            pltpu.sync_copy(ones_v, acc.at[idx_v], add=True)
            return carry

        lax.fori_loop(0, n_chunks, body, 0)
        plsc.subcore_barrier()
        pltpu.sync_copy(acc.at[pl.ds(rbase, rows_per_tile)],
                        out_hbm.at[c, pl.ds(rbase, rows_per_tile)])

    return k


def _agg_call(E, N, D):
    e_per_w = E // NW
    n_chunks = e_per_w // CHUNK
    rows_per_tile = _pad_rows(N)
    NP = rows_per_tile * NS

    @functools.partial(
        pl.kernel,
        mesh=_mesh(),
        out_type=jax.ShapeDtypeStruct((NC, NP, D), jnp.float32),
        compiler_params=pltpu.CompilerParams(use_tc_tiling_on_sc=False),
        scratch_types=[
            pltpu.VMEM((CHUNK,), jnp.int32),
            pltpu.VMEM((CHUNK,), jnp.int32),
            pltpu.VMEM((CHUNK, D), jnp.float32),
            pltpu.VMEM_SHARED((NP, D), jnp.float32),
            pltpu.SemaphoreType.DMA,
        ],
    )
    def k(src_hbm, dst_hbm, g_hbm, zeros_hbm, out_hbm,
          src_v, dst_v, rows_v, acc, sem):
        c = lax.axis_index("c")
        s = lax.axis_index("s")
        wid = c * NS + s
        rbase = s * rows_per_tile
        pltpu.sync_copy(zeros_hbm, acc.at[pl.ds(rbase, rows_per_tile)])
        plsc.subcore_barrier()
        ebase = wid * e_per_w

        def body(i, carry):
            off = ebase + i * CHUNK
            pltpu.sync_copy(src_hbm.at[pl.ds(off, CHUNK)], src_v)
            pltpu.sync_copy(dst_hbm.at[pl.ds(off, CHUNK)], dst_v)
            pltpu.async_copy(g_hbm.at[src_v], rows_v, sem).wait()
            pltpu.sync_copy(rows_v, acc.at[dst_v], add=True)
            return carry

        lax.fori_loop(0, n_chunks, body, 0)
        plsc.subcore_barrier()
        pltpu.sync_copy(acc.at[pl.ds(rbase, rows_per_tile)],
                        out_hbm.at[c, pl.ds(rbase, rows_per_tile)])

    return k


_ROWS = 2000


def _mm_scale(x, W, d0, d1):
    N, K = x.shape
    H = W.shape[1]

    def body(x_ref, w_ref, d0_ref, d1_ref, g_ref, dinv_ref):
        dinv = lax.rsqrt(d0_ref[...] + d1_ref[...] + 1.0)
        g_ref[...] = jnp.dot(x_ref[...], w_ref[...],
                             preferred_element_type=jnp.float32) * dinv
        dinv_ref[...] = dinv

    return pl.pallas_call(
        body,
        grid=(N // _ROWS,),
        in_specs=[
            pl.BlockSpec((_ROWS, K), lambda i: (i, 0)),
            pl.BlockSpec((K, H), lambda i: (0, 0)),
            pl.BlockSpec((_ROWS, 1), lambda i: (i, 0)),
            pl.BlockSpec((_ROWS, 1), lambda i: (i, 0)),
        ],
        out_specs=[
            pl.BlockSpec((_ROWS, H), lambda i: (i, 0)),
            pl.BlockSpec((_ROWS, 1), lambda i: (i, 0)),
        ],
        out_shape=[
            jax.ShapeDtypeStruct((N, H), jnp.float32),
            jax.ShapeDtypeStruct((N, 1), jnp.float32),
        ],
    )(x, W, d0, d1)


def _mid_layer(a0, a1, g1, dinv, b1, W2):
    N, H = g1.shape
    O = W2.shape[1]

    def body(a0_ref, a1_ref, g1_ref, dinv_ref, b1_ref, w2_ref, g2_ref):
        dinv = dinv_ref[...]
        out1 = dinv * (a0_ref[...] + a1_ref[...] + g1_ref[...]) + b1_ref[...]
        out1 = jnp.maximum(out1, 0.0)
        g2_ref[...] = jnp.dot(out1, w2_ref[...],
                              preferred_element_type=jnp.float32) * dinv

    return pl.pallas_call(
        body,
        grid=(N // _ROWS,),
        in_specs=[
            pl.BlockSpec((_ROWS, H), lambda i: (i, 0)),
            pl.BlockSpec((_ROWS, H), lambda i: (i, 0)),
            pl.BlockSpec((_ROWS, H), lambda i: (i, 0)),
            pl.BlockSpec((_ROWS, 1), lambda i: (i, 0)),
            pl.BlockSpec((1, H), lambda i: (0, 0)),
            pl.BlockSpec((H, O), lambda i: (0, 0)),
        ],
        out_specs=pl.BlockSpec((_ROWS, O), lambda i: (i, 0)),
        out_shape=jax.ShapeDtypeStruct((N, O), jnp.float32),
    )(a0, a1, g1, dinv, b1, W2)


def _final_layer(c0, c1, g2, dinv, b2):
    N, O = g2.shape

    def body(c0_ref, c1_ref, g2_ref, dinv_ref, b2_ref, o_ref):
        o_ref[...] = dinv_ref[...] * (c0_ref[...] + c1_ref[...] + g2_ref[...]) \
            + b2_ref[...]

    return pl.pallas_call(
        body,
        grid=(N // _ROWS,),
        in_specs=[
            pl.BlockSpec((_ROWS, O), lambda i: (i, 0)),
            pl.BlockSpec((_ROWS, O), lambda i: (i, 0)),
            pl.BlockSpec((_ROWS, O), lambda i: (i, 0)),
            pl.BlockSpec((_ROWS, 1), lambda i: (i, 0)),
            pl.BlockSpec((1, O), lambda i: (0, 0)),
        ],
        out_specs=pl.BlockSpec((_ROWS, O), lambda i: (i, 0)),
        out_shape=jax.ShapeDtypeStruct((N, O), jnp.float32),
    )(c0, c1, g2, dinv, b2)


_DEG_COLS = 8


def kernel(x, edge_index, W1, b1, W2, b2):
    N, _ = x.shape
    E = edge_index.shape[1]
    H = W1.shape[1]
    O = W2.shape[1]
    src = edge_index[0]
    dst = edge_index[1]

    rows_per_tile = _pad_rows(N)
    ones_c = jnp.ones((CHUNK, _DEG_COLS), jnp.float32)
    zeros_deg = jnp.zeros((rows_per_tile, _DEG_COLS), jnp.float32)
    deg_parts = _deg_call(E, N, _DEG_COLS)(dst, ones_c, zeros_deg)
    d0 = deg_parts[0, :N, :1]
    d1 = deg_parts[1, :N, :1]

    g1, dinv = _mm_scale(x, W1, d0, d1)

    zeros_h = jnp.zeros((rows_per_tile, H), jnp.float32)
    acc1 = _agg_call(E, N, H)(src, dst, g1, zeros_h)[:, :N]

    g2 = _mid_layer(acc1[0], acc1[1], g1, dinv,
                    b1.reshape(1, H), W2)

    zeros_o = jnp.zeros((rows_per_tile, O), jnp.float32)
    acc2 = _agg_call(E, N, O)(src, dst, g2, zeros_o)[:, :N]

    return _final_layer(acc2[0], acc2[1], g2, dinv, b2.reshape(1, O))


# speedup vs baseline: 43.1162x; 2.9286x over previous
import functools

import jax
import jax.numpy as jnp
from jax import lax
from jax.experimental import pallas as pl
from jax.experimental.pallas import tpu as pltpu
from jax.experimental.pallas import tpu_sc as plsc

NC = 2
NS = 16
NW = NC * NS
CHUNK = 125
NB = 4
DEG_K = 8


def _mesh():
    return plsc.VectorSubcoreMesh(core_axis_name="c", subcore_axis_name="s")


def _pad_rows(N):
    r = -(-N // NS)
    return -(-r // 8) * 8


def _deg_call(E, N):
    e_per_w = E // NW
    n_chunks = e_per_w // CHUNK
    rows_per_tile = _pad_rows(N)
    NP = rows_per_tile * NS

    @functools.partial(
        pl.kernel,
        mesh=_mesh(),
        out_type=jax.ShapeDtypeStruct((NC, NP, 1), jnp.float32),
        compiler_params=pltpu.CompilerParams(use_tc_tiling_on_sc=False),
        scratch_types=[
            pltpu.VMEM((n_chunks, CHUNK), jnp.int32),
            pltpu.VMEM((CHUNK, 1), jnp.float32),
            pltpu.VMEM_SHARED((NP, 1), jnp.float32),
            pltpu.SemaphoreType.DMA((DEG_K,)),
        ],
    )
    def k(dsts_hbm, ones_hbm, zeros_hbm, out_hbm, dst_v, ones_v, acc, sems):
        c = lax.axis_index("c")
        s = lax.axis_index("s")
        wid = c * NS + s
        rbase = s * rows_per_tile
        pltpu.sync_copy(zeros_hbm, acc.at[pl.ds(rbase, rows_per_tile)])
        pltpu.sync_copy(dsts_hbm.at[wid], dst_v)
        pltpu.sync_copy(ones_hbm, ones_v)
        plsc.subcore_barrier()

        def outer(o, carry):
            for b in range(DEG_K):
                i = o * DEG_K + b
                pltpu.async_copy(ones_v, acc.at[dst_v.at[i]], sems.at[b],
                                 add=True)
            for b in range(DEG_K):
                i = o * DEG_K + b
                pltpu.make_async_copy(ones_v, acc.at[dst_v.at[i]],
                                      sems.at[b]).wait()
            return carry

        lax.fori_loop(0, n_chunks // DEG_K, outer, 0)
        plsc.subcore_barrier()
        pltpu.sync_copy(acc.at[pl.ds(rbase, rows_per_tile)],
                        out_hbm.at[c, pl.ds(rbase, rows_per_tile)])

    return k


def _agg_call(E, N, D):
    e_per_w = E // NW
    n_chunks = e_per_w // CHUNK
    n_outer = n_chunks // NB
    rows_per_tile = _pad_rows(N)
    NP = rows_per_tile * NS

    @functools.partial(
        pl.kernel,
        mesh=_mesh(),
        out_type=jax.ShapeDtypeStruct((NC, NP, D), jnp.float32),
        compiler_params=pltpu.CompilerParams(use_tc_tiling_on_sc=False),
        scratch_types=[
            pltpu.VMEM((n_chunks, CHUNK), jnp.int32),
            pltpu.VMEM((n_chunks, CHUNK), jnp.int32),
            pltpu.VMEM((NB, CHUNK, D), jnp.float32),
            pltpu.VMEM_SHARED((NP, D), jnp.float32),
            pltpu.SemaphoreType.DMA((NB,)),
            pltpu.SemaphoreType.DMA((NB,)),
        ],
    )
    def k(srcs_hbm, dsts_hbm, g_hbm, zeros_hbm, out_hbm,
          src_v, dst_v, rows_v, acc, sem_g, sem_s):
        c = lax.axis_index("c")
        s = lax.axis_index("s")
        wid = c * NS + s
        rbase = s * rows_per_tile
        pltpu.sync_copy(zeros_hbm, acc.at[pl.ds(rbase, rows_per_tile)])
        pltpu.sync_copy(srcs_hbm.at[wid], src_v)
        pltpu.sync_copy(dsts_hbm.at[wid], dst_v)
        plsc.subcore_barrier()

        def start_gather(i, b):
            pltpu.async_copy(g_hbm.at[src_v.at[i]], rows_v.at[b], sem_g.at[b])

        def wait_gather(i, b):
            pltpu.make_async_copy(g_hbm.at[src_v.at[i]], rows_v.at[b],
                                  sem_g.at[b]).wait()

        def start_scatter(i, b):
            pltpu.async_copy(rows_v.at[b], acc.at[dst_v.at[i]], sem_s.at[b],
                             add=True)

        def wait_scatter(i, b):
            pltpu.make_async_copy(rows_v.at[b], acc.at[dst_v.at[i]],
                                  sem_s.at[b]).wait()

        for b in range(NB):
            start_gather(b, b)

        def outer(o, carry):
            for b in range(NB):
                i = o * NB + b
                wait_gather(i, b)
                start_scatter(i, b)
                wait_scatter(i, b)
                start_gather(i + NB, b)
            return carry

        lax.fori_loop(0, n_outer - 1, outer, 0)
        for b in range(NB):
            i = (n_outer - 1) * NB + b
            wait_gather(i, b)
            start_scatter(i, b)
        for b in range(NB):
            i = (n_outer - 1) * NB + b
            wait_scatter(i, b)

        plsc.subcore_barrier()
        pltpu.sync_copy(acc.at[pl.ds(rbase, rows_per_tile)],
                        out_hbm.at[c, pl.ds(rbase, rows_per_tile)])

    return k


_ROWS = 2000


def _matmul(x, W):
    N, K = x.shape
    H = W.shape[1]

    def body(x_ref, w_ref, h_ref):
        h_ref[...] = jnp.dot(x_ref[...], w_ref[...],
                             preferred_element_type=jnp.float32)

    return pl.pallas_call(
        body,
        grid=(N // _ROWS,),
        in_specs=[
            pl.BlockSpec((_ROWS, K), lambda i: (i, 0)),
            pl.BlockSpec((K, H), lambda i: (0, 0)),
        ],
        out_specs=pl.BlockSpec((_ROWS, H), lambda i: (i, 0)),
        out_shape=jax.ShapeDtypeStruct((N, H), jnp.float32),
    )(x, W)


def _scale_g(h, d0, d1):
    N, H = h.shape

    def body(h_ref, d0_ref, d1_ref, g_ref, dinv_ref):
        dinv = lax.rsqrt(d0_ref[...] + d1_ref[...] + 1.0)
        g_ref[...] = h_ref[...] * dinv
        dinv_ref[...] = dinv

    return pl.pallas_call(
        body,
        grid=(N // _ROWS,),
        in_specs=[
            pl.BlockSpec((_ROWS, H), lambda i: (i, 0)),
            pl.BlockSpec((_ROWS, 1), lambda i: (i, 0)),
            pl.BlockSpec((_ROWS, 1), lambda i: (i, 0)),
        ],
        out_specs=[
            pl.BlockSpec((_ROWS, H), lambda i: (i, 0)),
            pl.BlockSpec((_ROWS, 1), lambda i: (i, 0)),
        ],
        out_shape=[
            jax.ShapeDtypeStruct((N, H), jnp.float32),
            jax.ShapeDtypeStruct((N, 1), jnp.float32),
        ],
    )(h, d0, d1)


def _mid_layer(a0, a1, g1, dinv, b1, W2):
    N, H = g1.shape
    O = W2.shape[1]

    def body(a0_ref, a1_ref, g1_ref, dinv_ref, b1_ref, w2_ref, g2_ref):
        dinv = dinv_ref[...]
        out1 = dinv * (a0_ref[...] + a1_ref[...] + g1_ref[...]) + b1_ref[...]
        out1 = jnp.maximum(out1, 0.0)
        g2_ref[...] = jnp.dot(out1, w2_ref[...],
                              preferred_element_type=jnp.float32) * dinv

    return pl.pallas_call(
        body,
        grid=(N // _ROWS,),
        in_specs=[
            pl.BlockSpec((_ROWS, H), lambda i: (i, 0)),
            pl.BlockSpec((_ROWS, H), lambda i: (i, 0)),
            pl.BlockSpec((_ROWS, H), lambda i: (i, 0)),
            pl.BlockSpec((_ROWS, 1), lambda i: (i, 0)),
            pl.BlockSpec((1, H), lambda i: (0, 0)),
            pl.BlockSpec((H, O), lambda i: (0, 0)),
        ],
        out_specs=pl.BlockSpec((_ROWS, O), lambda i: (i, 0)),
        out_shape=jax.ShapeDtypeStruct((N, O), jnp.float32),
    )(a0, a1, g1, dinv, b1, W2)


def _final_layer(c0, c1, g2, dinv, b2):
    N, O = g2.shape

    def body(c0_ref, c1_ref, g2_ref, dinv_ref, b2_ref, o_ref):
        o_ref[...] = dinv_ref[...] * (c0_ref[...] + c1_ref[...] + g2_ref[...]) \
            + b2_ref[...]

    return pl.pallas_call(
        body,
        grid=(N // _ROWS,),
        in_specs=[
            pl.BlockSpec((_ROWS, O), lambda i: (i, 0)),
            pl.BlockSpec((_ROWS, O), lambda i: (i, 0)),
            pl.BlockSpec((_ROWS, O), lambda i: (i, 0)),
            pl.BlockSpec((_ROWS, 1), lambda i: (i, 0)),
            pl.BlockSpec((1, O), lambda i: (0, 0)),
        ],
        out_specs=pl.BlockSpec((_ROWS, O), lambda i: (i, 0)),
        out_shape=jax.ShapeDtypeStruct((N, O), jnp.float32),
    )(c0, c1, g2, dinv, b2)


def kernel(x, edge_index, W1, b1, W2, b2):
    N, _ = x.shape
    E = edge_index.shape[1]
    H = W1.shape[1]
    O = W2.shape[1]
    e_per_w = E // NW
    n_chunks = e_per_w // CHUNK
    srcs = edge_index[0].reshape(NW, n_chunks, CHUNK)
    dsts = edge_index[1].reshape(NW, n_chunks, CHUNK)

    rows_per_tile = _pad_rows(N)
    ones_c = jnp.ones((CHUNK, 1), jnp.float32)
    zeros_deg = jnp.zeros((rows_per_tile, 1), jnp.float32)
    deg_parts = _deg_call(E, N)(dsts, ones_c, zeros_deg)
    d0 = deg_parts[0, :N]
    d1 = deg_parts[1, :N]

    h1 = _matmul(x, W1)
    g1, dinv = _scale_g(h1, d0, d1)

    zeros_h = jnp.zeros((rows_per_tile, H), jnp.float32)
    acc1 = _agg_call(E, N, H)(srcs, dsts, g1, zeros_h)[:, :N]

    g2 = _mid_layer(acc1[0], acc1[1], g1, dinv,
                    b1.reshape(1, H), W2)

    zeros_o = jnp.zeros((rows_per_tile, O), jnp.float32)
    acc2 = _agg_call(E, N, O)(srcs, dsts, g2, zeros_o)[:, :N]

    return _final_layer(acc2[0], acc2[1], g2, dinv, b2.reshape(1, O))
